# baseline (speedup 1.0000x reference)
import functools

import jax
import jax.numpy as jnp
from jax import lax
from jax.experimental import pallas as pl
from jax.experimental.pallas import tpu as pltpu
from jax.experimental.pallas import tpu_sc as plsc

VOCAB = 1000000
EMBED = 16
BATCH = 16384
NEGS = 10

NC = 2
NS = 16
NW = NC * NS
CH = 128
CB = BATCH // NW // CH
NB = BATCH * NEGS // NW // CH


def _sc_gather_body(w_hbm, wctx_hbm, cidx_hbm, oidx_hbm, nidx_hbm,
               emc_hbm, emo_hbm, emn_hbm,
               cidx_v, oidx_v, nidx_v, crows, orows, nrows, sem_p, sem_n):
    wid = lax.axis_index("s") * NC + lax.axis_index("c")

    pltpu.sync_copy(cidx_hbm.at[pl.ds(wid * CB, CB)], cidx_v)
    pltpu.sync_copy(oidx_hbm.at[pl.ds(wid * CB, CB)], oidx_v)
    pltpu.sync_copy(nidx_hbm.at[pl.ds(wid * NB, NB)], nidx_v)

    pair = []
    for j in range(CB):
        pair.append(pltpu.async_copy(
            w_hbm.at[cidx_v.at[j]], crows.at[pl.ds(j * CH, CH)], sem_p))
        pair.append(pltpu.async_copy(
            wctx_hbm.at[oidx_v.at[j]], orows.at[pl.ds(j * CH, CH)], sem_p))

    STEP = 8

    @pl.loop(0, NB, step=STEP)
    def _neg_round(j):
        copies = [
            pltpu.async_copy(w_hbm.at[nidx_v.at[j + b]],
                             nrows.at[pl.ds((j + b) * CH, CH)], sem_n)
            for b in range(STEP)
        ]
        for c in copies:
            c.wait()

    for c in pair:
        c.wait()

    pltpu.sync_copy(crows, emc_hbm.at[pl.ds(wid * CB * CH, CB * CH)])
    pltpu.sync_copy(orows, emo_hbm.at[pl.ds(wid * CB * CH, CB * CH)])
    pltpu.sync_copy(nrows, emn_hbm.at[pl.ds(wid * NB * CH, NB * CH)])


@functools.cache
def _sc_gather():
    return pl.kernel(
        _sc_gather_body,
        out_type=(
            jax.ShapeDtypeStruct((BATCH, EMBED), jnp.float32),
            jax.ShapeDtypeStruct((BATCH, EMBED), jnp.float32),
            jax.ShapeDtypeStruct((BATCH * NEGS, EMBED), jnp.float32),
        ),
        mesh=plsc.VectorSubcoreMesh(core_axis_name="c", subcore_axis_name="s",
                                    num_cores=NC, num_subcores=NS),
        scratch_types=(
            pltpu.VMEM((CB, CH), jnp.int32),
            pltpu.VMEM((CB, CH), jnp.int32),
            pltpu.VMEM((NB, CH), jnp.int32),
            pltpu.VMEM((CB * CH, EMBED), jnp.float32),
            pltpu.VMEM((CB * CH, EMBED), jnp.float32),
            pltpu.VMEM((NB * CH, EMBED), jnp.float32),
            pltpu.SemaphoreType.DMA,
            pltpu.SemaphoreType.DMA,
        ),
        compiler_params=pltpu.CompilerParams(use_tc_tiling_on_sc=False),
    )


BS = 2048
GRID = BATCH // BS


def _tc_loss_body(cref, oref, nref, out_ref, acc_ref):
    i = pl.program_id(0)
    c = cref[...]
    o = oref[...]
    n = nref[...]

    def softplus(x):
        return jnp.maximum(x, 0.0) + jnp.log1p(jnp.exp(-jnp.abs(x)))

    pos = jnp.sum(c * o, axis=-1)
    bsum = jnp.sum(softplus(-pos))
    for k in range(NEGS):
        negd = jnp.sum(n[:, k, :] * c, axis=-1)
        bsum = bsum + jnp.sum(softplus(negd))

    @pl.when(i == 0)
    def _():
        acc_ref[0] = 0.0

    acc_ref[0] += bsum

    @pl.when(i == GRID - 1)
    def _():
        out_ref[0, 0] = acc_ref[0] / BATCH


_tc_loss = pl.pallas_call(
    _tc_loss_body,
    grid=(GRID,),
    in_specs=[
        pl.BlockSpec((BS, EMBED), lambda i: (i, 0)),
        pl.BlockSpec((BS, EMBED), lambda i: (i, 0)),
        pl.BlockSpec((BS, NEGS, EMBED), lambda i: (i, 0, 0)),
    ],
    out_specs=pl.BlockSpec(memory_space=pltpu.SMEM),
    out_shape=jax.ShapeDtypeStruct((1, 1), jnp.float32),
    scratch_shapes=[pltpu.SMEM((1,), jnp.float32)],
)


@jax.jit
def kernel(center_words, outside_words, neg_input_ids, W, W_ctx):
    cidx = center_words.astype(jnp.int32).reshape(BATCH // CH, CH)
    oidx = outside_words.astype(jnp.int32).reshape(BATCH // CH, CH)
    nidx = neg_input_ids.astype(jnp.int32).reshape(BATCH * NEGS // CH, CH)
    emc, emo, emn = _sc_gather()(W, W_ctx, cidx, oidx, nidx)
    emn3 = emn.reshape(BATCH, NEGS, EMBED)
    return _tc_loss(emc, emo, emn3)[0, 0]


# speedup vs baseline: 1.1940x; 1.1940x over previous
import functools

import jax
import jax.numpy as jnp
from jax import lax
from jax.experimental import pallas as pl
from jax.experimental.pallas import tpu as pltpu
from jax.experimental.pallas import tpu_sc as plsc

VOCAB = 1000000
EMBED = 16
BATCH = 16384
NEGS = 10

NC = 2
NS = 16
NW = NC * NS
CH = 128
BPW = BATCH // NW
CB = BPW // CH
SUPER = VOCAB // 8
NCH = NEGS * CB
XPW = CB + NCH
IR = 2 * CB + NCH


def _sc_body(wsup_hbm, ctxsup_hbm, sup_hbm, sub_hbm, x_hbm,
             sup_v, sub_v, bufa, bufb, ct_v, x_v, sem_a, sem_b):
    wid = lax.axis_index("s") * NC + lax.axis_index("c")

    pltpu.sync_copy(sup_hbm.at[wid], sup_v)
    pltpu.sync_copy(sub_hbm.at[wid], sub_v)

    lanes = lax.iota(jnp.int32, 16)

    def extract_center(chunk, buf):
        @pl.loop(0, CH // 16)
        def _grp(g):
            rows = g * 16 + lanes
            t = sub_v[chunk, pl.ds(g * 16, 16)]
            for d in range(EMBED):
                vec = plsc.load_gather(buf, [rows, t + d])
                ct_v[d, pl.ds(chunk * CH + g * 16, 16)] = vec

    def dot_chunk(srow, bchunk, xrow, negate, buf):
        @pl.loop(0, CH // 16)
        def _grp(g):
            rows = g * 16 + lanes
            t = sub_v[srow, pl.ds(g * 16, 16)]
            acc = jnp.zeros((16,), jnp.float32)
            for d in range(EMBED):
                vec = plsc.load_gather(buf, [rows, t + d])
                cvec = ct_v[d, pl.ds(bchunk * CH + g * 16, 16)]
                acc = acc + vec * cvec
            if negate:
                acc = -acc
            x_v[xrow, pl.ds(g * 16, 16)] = acc

    def wait_buf(buf, sem):
        pltpu.make_async_copy(wsup_hbm.at[pl.ds(0, CH)], buf, sem).wait()

    pltpu.async_copy(wsup_hbm.at[sup_v.at[0]], bufa, sem_a)
    pltpu.async_copy(wsup_hbm.at[sup_v.at[1]], bufb, sem_b)
    wait_buf(bufa, sem_a)
    extract_center(0, bufa)
    pltpu.async_copy(wsup_hbm.at[sup_v.at[2]], bufa, sem_a)
    wait_buf(bufb, sem_b)
    extract_center(1, bufb)
    pltpu.async_copy(wsup_hbm.at[sup_v.at[3]], bufb, sem_b)
    wait_buf(bufa, sem_a)
    extract_center(2, bufa)
    pltpu.async_copy(ctxsup_hbm.at[sup_v.at[CB + 0]], bufa, sem_a)
    wait_buf(bufb, sem_b)
    extract_center(3, bufb)
    pltpu.async_copy(ctxsup_hbm.at[sup_v.at[CB + 1]], bufb, sem_b)

    wait_buf(bufa, sem_a)
    dot_chunk(CB + 0, 0, 0, True, bufa)
    pltpu.async_copy(ctxsup_hbm.at[sup_v.at[CB + 2]], bufa, sem_a)
    wait_buf(bufb, sem_b)
    dot_chunk(CB + 1, 1, 1, True, bufb)
    pltpu.async_copy(ctxsup_hbm.at[sup_v.at[CB + 3]], bufb, sem_b)
    wait_buf(bufa, sem_a)
    dot_chunk(CB + 2, 2, 2, True, bufa)
    pltpu.async_copy(wsup_hbm.at[sup_v.at[2 * CB + 0]], bufa, sem_a)
    wait_buf(bufb, sem_b)
    dot_chunk(CB + 3, 3, 3, True, bufb)
    pltpu.async_copy(wsup_hbm.at[sup_v.at[2 * CB + 1]], bufb, sem_b)

    @pl.loop(0, NCH, step=2)
    def _neg(j):
        wait_buf(bufa, sem_a)
        dot_chunk(2 * CB + j, j & (CB - 1), CB + j, False, bufa)

        @pl.when(j + 2 < NCH)
        def _():
            pltpu.async_copy(wsup_hbm.at[sup_v.at[2 * CB + j + 2]], bufa,
                             sem_a)

        wait_buf(bufb, sem_b)
        dot_chunk(2 * CB + j + 1, (j + 1) & (CB - 1), CB + j + 1, False,
                  bufb)

        @pl.when(j + 3 < NCH)
        def _():
            pltpu.async_copy(wsup_hbm.at[sup_v.at[2 * CB + j + 3]], bufb,
                             sem_b)

    pltpu.sync_copy(x_v, x_hbm.at[wid])


@functools.cache
def _sc_dots():
    return pl.kernel(
        _sc_body,
        out_type=jax.ShapeDtypeStruct((NW, XPW, CH), jnp.float32),
        mesh=plsc.VectorSubcoreMesh(core_axis_name="c", subcore_axis_name="s",
                                    num_cores=NC, num_subcores=NS),
        scratch_types=(
            pltpu.VMEM((IR, CH), jnp.int32),
            pltpu.VMEM((IR, CH), jnp.int32),
            pltpu.VMEM((CH, CH), jnp.float32),
            pltpu.VMEM((CH, CH), jnp.float32),
            pltpu.VMEM((EMBED, BPW), jnp.float32),
            pltpu.VMEM((XPW, CH), jnp.float32),
            pltpu.SemaphoreType.DMA,
            pltpu.SemaphoreType.DMA,
        ),
        compiler_params=pltpu.CompilerParams(needs_layout_passes=False),
    )


def _tc_loss_body(xref, out_ref):
    x = xref[...]
    sp = jnp.maximum(x, 0.0) + jnp.log1p(jnp.exp(-jnp.abs(x)))
    out_ref[0, 0] = jnp.sum(sp) / BATCH


_tc_loss = pl.pallas_call(
    _tc_loss_body,
    out_specs=pl.BlockSpec(memory_space=pltpu.SMEM),
    out_shape=jax.ShapeDtypeStruct((1, 1), jnp.float32),
)


@jax.jit
def kernel(center_words, outside_words, neg_input_ids, W, W_ctx):
    wsup = W.reshape(SUPER, 128)
    ctxsup = W_ctx.reshape(SUPER, 128)
    c = center_words.astype(jnp.int32)
    o = outside_words.astype(jnp.int32)
    n = neg_input_ids.astype(jnp.int32).T

    def prep(v):
        return (v >> 3), ((v & 7) << 4)

    csup, csub = prep(c.reshape(NW, CB, CH))
    osup, osub = prep(o.reshape(NW, CB, CH))
    nw = n.reshape(NEGS, NW, CB, CH).transpose(1, 0, 2, 3).reshape(NW, NCH, CH)
    nsup, nsub = prep(nw)
    sup = jnp.concatenate([csup, osup, nsup], axis=1)
    sub = jnp.concatenate([csub, osub, nsub], axis=1)

    x = _sc_dots()(wsup, ctxsup, sup, sub)
    return _tc_loss(x)[0, 0]
